# baseline (speedup 1.0000x reference)
import jax
import jax.numpy as jnp
from jax.experimental import pallas as pl


def _split_body(zt_ref, a_ref, b_ref):
    ch = a_ref.shape[-1]
    a_ref[...] = zt_ref[:, :, :, :ch]
    b_ref[...] = zt_ref[:, :, :, ch:]


def kernel(z):
    n, c, h, w = z.shape
    ch = c // 2
    zt = jnp.transpose(z, (0, 2, 3, 1))

    hb = 8
    o1, o2 = pl.pallas_call(
        _split_body,
        grid=(n, h // hb),
        in_specs=[pl.BlockSpec((1, hb, w, c), lambda i, j: (i, j, 0, 0))],
        out_specs=[
            pl.BlockSpec((1, hb, w, ch), lambda i, j: (i, j, 0, 0)),
            pl.BlockSpec((1, hb, w, ch), lambda i, j: (i, j, 0, 0)),
        ],
        out_shape=[
            jax.ShapeDtypeStruct((n, h, w, ch), z.dtype),
            jax.ShapeDtypeStruct((n, h, w, ch), z.dtype),
        ],
    )(zt)

    z1 = jnp.transpose(o1, (0, 3, 1, 2))
    z2 = jnp.transpose(o2, (0, 3, 1, 2))
    log_det = jnp.zeros((), z.dtype)
    return (z1, z2, log_det)


# speedup vs baseline: 2.4597x; 2.4597x over previous
import jax
import jax.numpy as jnp
from jax.experimental import pallas as pl


def _split_body(zt_ref, a_ref, b_ref):
    ch = a_ref.shape[-1]
    a_ref[...] = zt_ref[:, :, :, :ch]
    b_ref[...] = zt_ref[:, :, :, ch:]


def kernel(z):
    n, c, h, w = z.shape
    ch = c // 2
    zt = jnp.transpose(z, (0, 2, 3, 1))

    nb = 2
    o1, o2 = pl.pallas_call(
        _split_body,
        grid=(n // nb,),
        in_specs=[pl.BlockSpec((nb, h, w, c), lambda i: (i, 0, 0, 0))],
        out_specs=[
            pl.BlockSpec((nb, h, w, ch), lambda i: (i, 0, 0, 0)),
            pl.BlockSpec((nb, h, w, ch), lambda i: (i, 0, 0, 0)),
        ],
        out_shape=[
            jax.ShapeDtypeStruct((n, h, w, ch), z.dtype),
            jax.ShapeDtypeStruct((n, h, w, ch), z.dtype),
        ],
    )(zt)

    z1 = jnp.transpose(o1, (0, 3, 1, 2))
    z2 = jnp.transpose(o2, (0, 3, 1, 2))
    log_det = jnp.zeros((), z.dtype)
    return (z1, z2, log_det)


# speedup vs baseline: 2.4628x; 1.0013x over previous
import jax
import jax.numpy as jnp
from jax.experimental import pallas as pl


def _split_body(zt_ref, a_ref, b_ref):
    ch = a_ref.shape[-1]
    a_ref[...] = zt_ref[:, :, :, :ch]
    b_ref[...] = zt_ref[:, :, :, ch:]


def kernel(z):
    n, c, h, w = z.shape
    ch = c // 2
    zt = jnp.transpose(z, (0, 2, 3, 1))

    nb = 4
    o1, o2 = pl.pallas_call(
        _split_body,
        grid=(n // nb,),
        in_specs=[pl.BlockSpec((nb, h, w, c), lambda i: (i, 0, 0, 0))],
        out_specs=[
            pl.BlockSpec((nb, h, w, ch), lambda i: (i, 0, 0, 0)),
            pl.BlockSpec((nb, h, w, ch), lambda i: (i, 0, 0, 0)),
        ],
        out_shape=[
            jax.ShapeDtypeStruct((n, h, w, ch), z.dtype),
            jax.ShapeDtypeStruct((n, h, w, ch), z.dtype),
        ],
    )(zt)

    z1 = jnp.transpose(o1, (0, 3, 1, 2))
    z2 = jnp.transpose(o2, (0, 3, 1, 2))
    log_det = jnp.zeros((), z.dtype)
    return (z1, z2, log_det)
